# baseline (speedup 1.0000x reference)
import functools

import jax
import jax.numpy as jnp
from jax.experimental import pallas as pl
from jax.experimental.pallas import tpu as pltpu

N = 10000
IN_F = 128
HID = 128
OUT_F = 64
BR = 400


def _gc1_body(x_ref, W1_ref, b1_ref, W2_ref, adj_ref, s2_ref, s1_scr):
    @pl.when(pl.program_id(0) == 0)
    def _():
        s1_scr[...] = jnp.dot(x_ref[...], W1_ref[...],
                              preferred_element_type=jnp.float32)

    h = jnp.dot(adj_ref[...], s1_scr[...],
                preferred_element_type=jnp.float32) + b1_ref[...]
    h = jnp.where(h >= 0, h, 0.2 * h)
    s2_ref[...] = jnp.dot(h, W2_ref[...], preferred_element_type=jnp.float32)


def _gc2_body(s2_ref, b2_ref, adj_ref, out_ref):
    o = jnp.dot(adj_ref[...], s2_ref[...],
                preferred_element_type=jnp.float32) + b2_ref[...]
    m = jnp.max(o, axis=1, keepdims=True)
    e = o - m
    lse = jnp.log(jnp.sum(jnp.exp(e), axis=1, keepdims=True))
    out_ref[...] = e - lse


@functools.partial(jax.jit, static_argnames=())
def kernel(x, adj, W1, b1, W2, b2):
    nb = N // BR
    b1r = b1.reshape(1, HID)
    b2r = b2.reshape(1, OUT_F)

    support2 = pl.pallas_call(
        _gc1_body,
        grid=(nb,),
        in_specs=[
            pl.BlockSpec((N, IN_F), lambda i: (0, 0)),
            pl.BlockSpec((IN_F, HID), lambda i: (0, 0)),
            pl.BlockSpec((1, HID), lambda i: (0, 0)),
            pl.BlockSpec((HID, OUT_F), lambda i: (0, 0)),
            pl.BlockSpec((BR, N), lambda i: (i, 0)),
        ],
        out_specs=pl.BlockSpec((BR, OUT_F), lambda i: (i, 0)),
        out_shape=jax.ShapeDtypeStruct((N, OUT_F), jnp.float32),
        scratch_shapes=[pltpu.VMEM((N, HID), jnp.float32)],
        compiler_params=pltpu.CompilerParams(
            dimension_semantics=("arbitrary",),
        ),
    )(x, W1, b1r, W2, adj)

    out = pl.pallas_call(
        _gc2_body,
        grid=(nb,),
        in_specs=[
            pl.BlockSpec((N, OUT_F), lambda i: (0, 0)),
            pl.BlockSpec((1, OUT_F), lambda i: (0, 0)),
            pl.BlockSpec((BR, N), lambda i: (i, 0)),
        ],
        out_specs=pl.BlockSpec((BR, OUT_F), lambda i: (i, 0)),
        out_shape=jax.ShapeDtypeStruct((N, OUT_F), jnp.float32),
        compiler_params=pltpu.CompilerParams(
            dimension_semantics=("arbitrary",),
        ),
    )(support2, b2r, adj)

    return out


# speedup vs baseline: 1.0022x; 1.0022x over previous
import functools

import jax
import jax.numpy as jnp
from jax.experimental import pallas as pl
from jax.experimental.pallas import tpu as pltpu

N = 10000
IN_F = 128
HID = 128
OUT_F = 64
BR = 400


def _bdot(a, b):
    return jnp.dot(a.astype(jnp.bfloat16), b.astype(jnp.bfloat16),
                   preferred_element_type=jnp.float32)


def _gc1_body(x_ref, W1_ref, b1_ref, W2_ref, adj_ref, s2_ref, s1_scr):
    @pl.when(pl.program_id(0) == 0)
    def _():
        s1_scr[...] = jnp.dot(x_ref[...], W1_ref[...],
                              preferred_element_type=jnp.float32)

    h = _bdot(adj_ref[...], s1_scr[...]) + b1_ref[...]
    h = jnp.where(h >= 0, h, 0.2 * h)
    s2_ref[...] = _bdot(h, W2_ref[...])


def _gc2_body(s2_ref, b2_ref, adj_ref, out_ref):
    o = _bdot(adj_ref[...], s2_ref[...]) + b2_ref[...]
    m = jnp.max(o, axis=1, keepdims=True)
    e = o - m
    lse = jnp.log(jnp.sum(jnp.exp(e), axis=1, keepdims=True))
    out_ref[...] = e - lse


@functools.partial(jax.jit, static_argnames=())
def kernel(x, adj, W1, b1, W2, b2):
    nb = N // BR
    b1r = b1.reshape(1, HID)
    b2r = b2.reshape(1, OUT_F)

    support2 = pl.pallas_call(
        _gc1_body,
        grid=(nb,),
        in_specs=[
            pl.BlockSpec((N, IN_F), lambda i: (0, 0)),
            pl.BlockSpec((IN_F, HID), lambda i: (0, 0)),
            pl.BlockSpec((1, HID), lambda i: (0, 0)),
            pl.BlockSpec((HID, OUT_F), lambda i: (0, 0)),
            pl.BlockSpec((BR, N), lambda i: (i, 0)),
        ],
        out_specs=pl.BlockSpec((BR, OUT_F), lambda i: (i, 0)),
        out_shape=jax.ShapeDtypeStruct((N, OUT_F), jnp.float32),
        scratch_shapes=[pltpu.VMEM((N, HID), jnp.float32)],
        compiler_params=pltpu.CompilerParams(
            dimension_semantics=("arbitrary",),
        ),
    )(x, W1, b1r, W2, adj)

    out = pl.pallas_call(
        _gc2_body,
        grid=(nb,),
        in_specs=[
            pl.BlockSpec((N, OUT_F), lambda i: (0, 0)),
            pl.BlockSpec((1, OUT_F), lambda i: (0, 0)),
            pl.BlockSpec((BR, N), lambda i: (i, 0)),
        ],
        out_specs=pl.BlockSpec((BR, OUT_F), lambda i: (i, 0)),
        out_shape=jax.ShapeDtypeStruct((N, OUT_F), jnp.float32),
        compiler_params=pltpu.CompilerParams(
            dimension_semantics=("arbitrary",),
        ),
    )(support2, b2r, adj)

    return out


# speedup vs baseline: 1.0298x; 1.0276x over previous
import jax
import jax.numpy as jnp
from jax.experimental import pallas as pl
from jax.experimental.pallas import tpu as pltpu

N = 10000
IN_F = 128
HID = 128
OUT_F = 64
BR = 400
NB = N // BR


def _body(x_ref, W1_ref, b1_ref, W2_ref, b2_ref, adj_ref, out_ref,
          s1_scr, s2f_scr, s2_scr):
    p = pl.program_id(0)

    @pl.when(p == 0)
    def _phase_a():
        @pl.when(pl.program_id(1) == 0)
        def _():
            s1_scr[...] = jnp.dot(
                x_ref[...], W1_ref[...],
                preferred_element_type=jnp.float32).astype(jnp.bfloat16)

        i = pl.program_id(1)
        adj_bf = adj_ref[...].astype(jnp.bfloat16)
        h = jnp.dot(adj_bf, s1_scr[...],
                    preferred_element_type=jnp.float32) + b1_ref[...]
        h = jnp.where(h >= 0, h, 0.2 * h)
        s2f_scr[pl.ds(i * BR, BR), :] = jnp.dot(
            h.astype(jnp.bfloat16), W2_ref[...],
            preferred_element_type=jnp.float32)

    @pl.when(p == 1)
    def _phase_b():
        @pl.when(pl.program_id(1) == 0)
        def _():
            s2_scr[...] = s2f_scr[...].astype(jnp.bfloat16)

        adj_bf = adj_ref[...].astype(jnp.bfloat16)
        o = jnp.dot(adj_bf, s2_scr[...],
                    preferred_element_type=jnp.float32) + b2_ref[...]
        m = jnp.max(o, axis=1, keepdims=True)
        e = o - m
        lse = jnp.log(jnp.sum(jnp.exp(e), axis=1, keepdims=True))
        out_ref[...] = e - lse


def kernel(x, adj, W1, b1, W2, b2):
    return pl.pallas_call(
        _body,
        grid=(2, NB),
        in_specs=[
            pl.BlockSpec((N, IN_F), lambda p, i: (0, 0)),
            pl.BlockSpec((IN_F, HID), lambda p, i: (0, 0)),
            pl.BlockSpec((1, HID), lambda p, i: (0, 0)),
            pl.BlockSpec((HID, OUT_F), lambda p, i: (0, 0)),
            pl.BlockSpec((1, OUT_F), lambda p, i: (0, 0)),
            pl.BlockSpec((BR, N), lambda p, i: (i, 0)),
        ],
        out_specs=pl.BlockSpec((BR, OUT_F), lambda p, i: (p * i, 0)),
        out_shape=jax.ShapeDtypeStruct((N, OUT_F), jnp.float32),
        scratch_shapes=[
            pltpu.VMEM((N, HID), jnp.bfloat16),
            pltpu.VMEM((N, OUT_F), jnp.float32),
            pltpu.VMEM((N, OUT_F), jnp.bfloat16),
        ],
        compiler_params=pltpu.CompilerParams(
            dimension_semantics=("arbitrary", "arbitrary"),
        ),
    )(x, W1, b1.reshape(1, HID), W2.astype(jnp.bfloat16),
      b2.reshape(1, OUT_F), adj)
